# baseline (speedup 1.0000x reference)
import functools

import jax
import jax.numpy as jnp
from jax import lax
from jax.experimental import pallas as pl
from jax.experimental.pallas import tpu as pltpu
from jax.experimental.pallas import tpu_sc as plsc

B = 16384
DIM = 16
EPS = 1e-6

_NC = 2
_NS = 16
_NW = _NC * _NS
_BPW = B // _NW
_CH = 128
_NCH = _BPW // _CH


def _gather_body(idx1_hbm, idx2_hbm, t1_hbm, t2_hbm, e1_hbm, e2_hbm,
                 idx1_v, idx2_v, r1_v, r2_v, sem):
    c = lax.axis_index("c")
    s = lax.axis_index("s")
    wid = s * _NC + c
    base = wid * _BPW
    row0 = wid * _NCH
    pltpu.sync_copy(idx1_hbm.at[pl.ds(row0, _NCH)], idx1_v)
    pltpu.sync_copy(idx2_hbm.at[pl.ds(row0, _NCH)], idx2_v)
    copies = []
    for k in range(_NCH):
        copies.append(pltpu.async_copy(
            t1_hbm.at[idx1_v.at[k]], r1_v.at[pl.ds(k * _CH, _CH)], sem))
        copies.append(pltpu.async_copy(
            t2_hbm.at[idx2_v.at[k]], r2_v.at[pl.ds(k * _CH, _CH)], sem))
    for cp in copies:
        cp.wait()
    pltpu.sync_copy(r1_v, e1_hbm.at[pl.ds(base, _BPW)])
    pltpu.sync_copy(r2_v, e2_hbm.at[pl.ds(base, _BPW)])


_gather = pl.kernel(
    _gather_body,
    out_type=(
        jax.ShapeDtypeStruct((B, DIM), jnp.float32),
        jax.ShapeDtypeStruct((B, DIM), jnp.float32),
    ),
    mesh=plsc.VectorSubcoreMesh(core_axis_name="c", subcore_axis_name="s"),
    compiler_params=pltpu.CompilerParams(use_tc_tiling_on_sc=False),
    scratch_types=[
        pltpu.VMEM((_NCH, _CH), jnp.int32),
        pltpu.VMEM((_NCH, _CH), jnp.int32),
        pltpu.VMEM((_BPW, DIM), jnp.float32),
        pltpu.VMEM((_BPW, DIM), jnp.float32),
        pltpu.SemaphoreType.DMA,
    ],
)


def _math_body(e1_ref, e2_ref, out_ref):
    e1 = e1_ref[...]
    e2 = e2_ref[...]
    dot = jnp.sum(e1 * e2, axis=1)
    s1 = jnp.sum(e1 * e1, axis=1)
    s2 = jnp.sum(e2 * e2, axis=1)
    cos = dot / jnp.maximum(jnp.sqrt(s1) * jnp.sqrt(s2), EPS)
    x = 100.0 * cos
    out_ref[...] = jnp.minimum(x, 0.0) - jnp.log1p(jnp.exp(-jnp.abs(x)))


_math = pl.pallas_call(
    _math_body,
    out_shape=jax.ShapeDtypeStruct((B,), jnp.float32),
)


def kernel(idx1, idx2, emb1, emb2, table1, table2):
    del emb1, emb2
    e1, e2 = _gather(idx1.reshape(B // _CH, _CH), idx2.reshape(B // _CH, _CH),
                     table1, table2)
    return _math(e1, e2)


# speedup vs baseline: 6.3390x; 6.3390x over previous
import jax
import jax.numpy as jnp
from jax import lax
from jax.experimental import pallas as pl
from jax.experimental.pallas import tpu as pltpu
from jax.experimental.pallas import tpu_sc as plsc

B = 16384
DIM = 16
EPS = 1e-6

_NC = 2
_NS = 16
_NW = _NC * _NS
_BPW = B // _NW
_G = 16
_NG = _BPW // _G
_CH = 128
_R = B // _CH


def _sc_body(idx1_hbm, idx2_hbm, t1_hbm, t2_hbm,
             dot_hbm, s1_hbm, s2_hbm,
             idx1_v, idx2_v, buf1_v, buf2_v,
             dot_v, s1_v, s2_v, sem, sem2):
    c = lax.axis_index("c")
    s = lax.axis_index("s")
    wid = s * _NC + c
    base = wid * _BPW

    cp3 = pltpu.async_copy(idx1_hbm.at[pl.ds(base, _BPW)], idx1_v, sem2)
    cp4 = pltpu.async_copy(idx2_hbm.at[pl.ds(base, _BPW)], idx2_v, sem2)
    cp3.wait()
    cp4.wait()

    lanes = lax.iota(jnp.int32, 16)

    @pl.loop(0, _NG)
    def _group(g):
        r0 = g * _G
        sl = pl.ds(r0, _G)
        iv1 = idx1_v[sl]
        iv2 = idx2_v[sl]
        cv1 = (iv1 >> 7) * 128
        cv2 = (iv2 >> 7) * 128
        copies = []
        for l in range(_G):
            c1 = pl.multiple_of(cv1[l], 128)
            c2 = pl.multiple_of(cv2[l], 128)
            copies.append(pltpu.async_copy(
                t1_hbm.at[:, pl.ds(c1, 128)], buf1_v.at[l], sem))
            copies.append(pltpu.async_copy(
                t2_hbm.at[:, pl.ds(c2, 128)], buf2_v.at[l], sem))
        for cp in copies:
            cp.wait()

        sub1 = iv1 & 127
        sub2 = iv2 & 127
        dot = jnp.zeros((16,), jnp.float32)
        s1 = jnp.zeros((16,), jnp.float32)
        s2 = jnp.zeros((16,), jnp.float32)
        for d in range(DIM):
            dv = jnp.full((16,), d, jnp.int32)
            v1 = plsc.load_gather(buf1_v, [lanes, dv, sub1])
            v2 = plsc.load_gather(buf2_v, [lanes, dv, sub2])
            dot = dot + v1 * v2
            s1 = s1 + v1 * v1
            s2 = s2 + v2 * v2
        dot_v[sl] = dot
        s1_v[sl] = s1
        s2_v[sl] = s2

    pltpu.sync_copy(dot_v, dot_hbm.at[pl.ds(base, _BPW)])
    pltpu.sync_copy(s1_v, s1_hbm.at[pl.ds(base, _BPW)])
    pltpu.sync_copy(s2_v, s2_hbm.at[pl.ds(base, _BPW)])


_sc_reduce = pl.kernel(
    _sc_body,
    out_type=(
        jax.ShapeDtypeStruct((B,), jnp.float32),
        jax.ShapeDtypeStruct((B,), jnp.float32),
        jax.ShapeDtypeStruct((B,), jnp.float32),
    ),
    mesh=plsc.VectorSubcoreMesh(core_axis_name="c", subcore_axis_name="s"),
    compiler_params=pltpu.CompilerParams(needs_layout_passes=False),
    scratch_types=[
        pltpu.VMEM((_BPW,), jnp.int32),
        pltpu.VMEM((_BPW,), jnp.int32),
        pltpu.VMEM((_G, DIM, _CH), jnp.float32),
        pltpu.VMEM((_G, DIM, _CH), jnp.float32),
        pltpu.VMEM((_BPW,), jnp.float32),
        pltpu.VMEM((_BPW,), jnp.float32),
        pltpu.VMEM((_BPW,), jnp.float32),
        pltpu.SemaphoreType.DMA,
        pltpu.SemaphoreType.DMA,
    ],
)


def _math_body(dot_ref, s1_ref, s2_ref, out_ref):
    dot = dot_ref[...]
    s1 = s1_ref[...]
    s2 = s2_ref[...]
    cos = dot / jnp.maximum(jnp.sqrt(s1) * jnp.sqrt(s2), EPS)
    x = 100.0 * cos
    out_ref[...] = jnp.minimum(x, 0.0) - jnp.log1p(jnp.exp(-jnp.abs(x)))


_math = pl.pallas_call(
    _math_body,
    out_shape=jax.ShapeDtypeStruct((_R, _CH), jnp.float32),
)


def kernel(idx1, idx2, emb1, emb2, table1, table2):
    del emb1, emb2
    dot, s1, s2 = _sc_reduce(idx1, idx2, table1.T, table2.T)
    return _math(dot.reshape(_R, _CH), s1.reshape(_R, _CH),
                 s2.reshape(_R, _CH)).reshape(B)
